# baseline (speedup 1.0000x reference)
import functools

import jax
import jax.numpy as jnp
from jax import lax
from jax.experimental import pallas as pl
from jax.experimental.pallas import tpu as pltpu
from jax.experimental.pallas import tpu_sc as plsc

_N = 10000
_E = 320000
_D = 128
_NC = 2
_NS = 16
_NW = _NC * _NS
_CH = 80
_EPW = _E // _NW
_NCHUNK = _EPW // _CH
_SG = 25
_NSG = _NCHUNK // _SG
_RPT = 632
_NPAD = _RPT * _NS
_DW = _D // 2


def _spmm_body(tbl_hbm, src_hbm, dst_hbm, w_hbm, out_hbm,
               src_v, dst_v, w_v, g0, g1, f0, f1, acc_sh, gsem, ssem):
    cid = lax.axis_index("c")
    sid = lax.axis_index("s")
    wid = cid * _NS + sid
    gbufs = (g0, g1)
    fbufs = (f0, f1)

    def _zrow(i, carry):
        for k in range(_D // 16):
            f0[i, pl.ds(k * 16, 16)] = jnp.zeros((16,), jnp.float32)
        return carry
    lax.fori_loop(0, _CH, _zrow, None)
    r0 = sid * _RPT
    for t in range(7):
        pltpu.sync_copy(f0, acc_sh.at[pl.ds(r0 + t * _CH, _CH), :])
    pltpu.sync_copy(f0.at[pl.ds(0, _RPT - 7 * _CH), :],
                    acc_sh.at[pl.ds(r0 + 7 * _CH, _RPT - 7 * _CH), :])
    plsc.subcore_barrier()

    def _gather(j, gb):
        pltpu.async_copy(tbl_hbm.at[src_v.at[pl.ds(j * _CH, _CH)]], gb, gsem)

    def _wait_gather(j, gb):
        pltpu.make_async_copy(
            tbl_hbm.at[src_v.at[pl.ds(j * _CH, _CH)]], gb, gsem).wait()

    def _scatter(j, fb):
        pltpu.async_copy(fb, acc_sh.at[dst_v.at[j]], ssem, add=True)

    def _wait_scatter(j, fb):
        pltpu.make_async_copy(fb, acc_sh.at[dst_v.at[j]], ssem).wait()

    def _scale(j, gb, fb):
        def _g16(i, c2):
            wv = w_v[pl.ds(j * _CH + i * 16, 16)]
            for jj in range(16):
                sp = wv.at[jnp.full((16,), jj, jnp.int32)].get(
                    mode="promise_in_bounds")
                e = i * 16 + jj
                c16 = jnp.full((16,), 16, jnp.int32)
                cmask = jnp.full((16,), -65536, jnp.int32)
                for b in range(_D // 32):
                    wrd = gb[e, pl.ds(b * 16, 16)]
                    lo = lax.bitcast_convert_type(
                        lax.shift_left(wrd, c16), jnp.float32)
                    hi = lax.bitcast_convert_type(
                        lax.bitwise_and(wrd, cmask), jnp.float32)
                    fb[e, pl.ds(b * 32, 16)] = lo * sp
                    fb[e, pl.ds(b * 32 + 16, 16)] = hi * sp
            return c2
        lax.fori_loop(0, _CH // 16, _g16, None)

    def _step(j, par):
        @pl.when(j >= 2)
        def _():
            _wait_scatter(j - 2, fbufs[par])
        _wait_gather(j, gbufs[par])
        _scale(j, gbufs[par], fbufs[par])
        _scatter(j, fbufs[par])

    def _group(s, carry):
        pltpu.sync_copy(src_hbm.at[wid, s], src_v)
        pltpu.sync_copy(dst_hbm.at[wid, s], dst_v)
        pltpu.sync_copy(w_hbm.at[wid, s], w_v)
        _gather(0, g0)

        def _pair(t, c1):
            for par in range(2):
                j = 2 * t + par
                _gather(j + 1, gbufs[1 - par])
                _step(j, par)
            return c1
        lax.fori_loop(0, (_SG - 1) // 2, _pair, None)

        _step(_SG - 1, (_SG - 1) % 2)
        for j in range(_SG - 2, _SG):
            _wait_scatter(j, fbufs[j % 2])
        return carry
    lax.fori_loop(0, _NSG, _group, None)

    plsc.subcore_barrier()

    @pl.when(sid < _NS - 1)
    def _dump_full():
        pltpu.sync_copy(acc_sh.at[pl.ds(r0, _RPT), :],
                        out_hbm.at[pl.ds(cid * _N + r0, _RPT), :])

    @pl.when(sid == _NS - 1)
    def _dump_tail():
        rem = _N - (_NS - 1) * _RPT
        pltpu.sync_copy(acc_sh.at[pl.ds(r0, rem), :],
                        out_hbm.at[pl.ds(cid * _N + r0, rem), :])


def _make_spmm():
    mesh = plsc.VectorSubcoreMesh(core_axis_name="c", subcore_axis_name="s")
    return pl.kernel(
        _spmm_body,
        out_type=jax.ShapeDtypeStruct((_NC * _N, _D), jnp.float32),
        mesh=mesh,
        scratch_types=[
            pltpu.VMEM((_SG * _CH,), jnp.int32),
            pltpu.VMEM((_SG, _CH), jnp.int32),
            pltpu.VMEM((_SG * _CH,), jnp.float32),
            pltpu.VMEM((_CH, _DW), jnp.int32),
            pltpu.VMEM((_CH, _DW), jnp.int32),
            pltpu.VMEM((_CH, _D), jnp.float32),
            pltpu.VMEM((_CH, _D), jnp.float32),
            pltpu.VMEM_SHARED((_NPAD, _D), jnp.float32),
            pltpu.SemaphoreType.DMA,
            pltpu.SemaphoreType.DMA,
        ],
        compiler_params=pltpu.CompilerParams(use_tc_tiling_on_sc=False),
    )


def _film_body(p0, p1, seqb, wfc, nt, wg, bg, wb, bb, bias, a, out):
    dims = (((1,), (1,)), ((), ()))
    x = p0[...] + p1[...]
    agg = lax.dot_general(x, wfc[...], dims,
                          preferred_element_type=jnp.float32)
    fts = lax.dot_general(seqb[...], wfc[...], dims,
                          preferred_element_type=jnp.float32)
    is0 = nt[...] == 0
    gam = jnp.where(is0, wg[0:1, :], wg[1:2, :]) + bg[...]
    bet = jnp.where(is0, wb[0:1, :], wb[1:2, :]) + bb[...]
    y = gam * agg + bet + bias[...] + fts
    aa = a[0, 0]
    out[...] = jnp.where(y >= 0.0, y, aa * y)


_BM = 1000


def _make_film():
    nb = _N // _BM
    row_spec = pl.BlockSpec((_BM, _D), lambda i: (i, 0))
    full = lambda shape: pl.BlockSpec(shape, lambda i: (0, 0))
    return pl.pallas_call(
        _film_body,
        grid=(nb,),
        in_specs=[
            row_spec,
            pl.BlockSpec((_BM, _D), lambda i: (i + nb, 0)),
            row_spec,
            full((_D, _D)),
            pl.BlockSpec((_BM, 1), lambda i: (i, 0)),
            full((_NC, _D)),
            full((1, _D)),
            full((_NC, _D)),
            full((1, _D)),
            full((1, _D)),
            pl.BlockSpec(memory_space=pltpu.SMEM),
        ],
        out_specs=row_spec,
        out_shape=jax.ShapeDtypeStruct((_N, _D), jnp.float32),
    )


def kernel(seq, edge_index, edge_weight, node_type, W_fc, W_gamma, b_gamma,
           W_beta, b_beta, bias, prelu_a):
    src = edge_index[0].reshape(_NW, _NSG, _SG * _CH)
    dst = edge_index[1].reshape(_NW, _NSG, _SG, _CH)
    w2 = edge_weight.reshape(_NW, _NSG, _SG * _CH)

    sb = seq.astype(jnp.bfloat16).reshape(_N, _D // 32, 2, 16)
    sb = sb.swapaxes(2, 3).reshape(_N, _DW, 2)
    tbl = lax.bitcast_convert_type(sb, jnp.int32)

    partials = _make_spmm()(tbl, src, dst, w2)

    return _make_film()(
        partials, partials, seq, W_fc,
        node_type.reshape(_N, 1),
        W_gamma.T, b_gamma.reshape(1, _D),
        W_beta.T, b_beta.reshape(1, _D),
        bias.reshape(1, _D),
        prelu_a.reshape(1, 1),
    )


# speedup vs baseline: 1.7317x; 1.7317x over previous
import functools

import jax
import jax.numpy as jnp
from jax import lax
from jax.experimental import pallas as pl
from jax.experimental.pallas import tpu as pltpu
from jax.experimental.pallas import tpu_sc as plsc

_N = 10000
_E = 320000
_D = 128
_NC = 2
_NS = 16
_NW = _NC * _NS
_CH = 80
_EPW = _E // _NW
_NCHUNK = _EPW // _CH
_SG = 25
_NSG = _NCHUNK // _SG
_RPT = 640
_NPAD = _RPT * _NS
_ZR = 80


def _spmm_body(seq_hbm, src_hbm, dst_hbm, w_hbm, out_hbm,
               src_v, dst_v, w_v, rows0, rows1, rows2, acc_sh, gsem, ssem):
    cid = lax.axis_index("c")
    sid = lax.axis_index("s")
    wid = cid * _NS + sid
    bufs = (rows0, rows1, rows2)

    def _zrow(i, carry):
        for k in range(_D // 16):
            rows0[i, pl.ds(k * 16, 16)] = jnp.zeros((16,), jnp.float32)
        return carry
    lax.fori_loop(0, _ZR, _zrow, None)
    r0 = sid * _RPT
    for t in range(_RPT // _ZR):
        pltpu.sync_copy(rows0, acc_sh.at[pl.ds(r0 + t * _ZR, _ZR), :])
    plsc.subcore_barrier()

    def _gather(j, b):
        pltpu.async_copy(seq_hbm.at[src_v.at[j]], bufs[b], gsem)

    def _wait_gather(j, b):
        pltpu.make_async_copy(seq_hbm.at[src_v.at[j]], bufs[b], gsem).wait()

    def _scatter(j, b):
        pltpu.async_copy(bufs[b], acc_sh.at[dst_v.at[j]], ssem, add=True)

    def _wait_scatter(j, b):
        pltpu.make_async_copy(bufs[b], acc_sh.at[dst_v.at[j]], ssem).wait()

    def _scale(j, b):
        rows_b = bufs[b]

        def _g16(i, c2):
            wv = w_v[j, pl.ds(i * 16, 16)]
            for jj in range(16):
                sp = wv.at[jnp.full((16,), jj, jnp.int32)].get(
                    mode="promise_in_bounds")
                e = i * 16 + jj
                for k in range(_D // 16):
                    rows_b[e, pl.ds(k * 16, 16)] = (
                        rows_b[e, pl.ds(k * 16, 16)] * sp)
            return c2
        lax.fori_loop(0, _CH // 16, _g16, None)

    def _group(s, carry):
        pltpu.sync_copy(src_hbm.at[wid, s], src_v)
        pltpu.sync_copy(dst_hbm.at[wid, s], dst_v)
        pltpu.sync_copy(w_hbm.at[wid, s], w_v)
        _gather(0, 0)
        _gather(1, 1)

        def _triple(t, c1):
            for jj in range(3):
                j = 3 * t + jj
                nb = (jj + 2) % 3

                @pl.when(j >= 1)
                def _():
                    _wait_scatter(j - 1, nb)
                _gather(j + 2, nb)
                _wait_gather(j, jj)
                _scale(j, jj)
                _scatter(j, jj)
            return c1
        lax.fori_loop(0, (_SG - 4) // 3, _triple, None)

        for j in range(_SG - 4, _SG):
            b = j % 3
            if j + 2 < _SG:
                _wait_scatter(j - 1, (j + 2) % 3)
                _gather(j + 2, (j + 2) % 3)
            _wait_gather(j, b)
            _scale(j, b)
            _scatter(j, b)
        for j in range(_SG - 3, _SG):
            _wait_scatter(j, j % 3)
        return carry
    lax.fori_loop(0, _NSG, _group, None)

    plsc.subcore_barrier()

    @pl.when(sid < _NS - 1)
    def _dump_full():
        pltpu.sync_copy(acc_sh.at[pl.ds(r0, _RPT), :],
                        out_hbm.at[pl.ds(cid * _N + r0, _RPT), :])

    @pl.when(sid == _NS - 1)
    def _dump_tail():
        rem = _N - (_NS - 1) * _RPT
        pltpu.sync_copy(acc_sh.at[pl.ds(r0, rem), :],
                        out_hbm.at[pl.ds(cid * _N + r0, rem), :])


def _make_spmm():
    mesh = plsc.VectorSubcoreMesh(core_axis_name="c", subcore_axis_name="s")
    return pl.kernel(
        _spmm_body,
        out_type=jax.ShapeDtypeStruct((_NC * _N, _D), jnp.float32),
        mesh=mesh,
        scratch_types=[
            pltpu.VMEM((_SG, _CH), jnp.int32),
            pltpu.VMEM((_SG, _CH), jnp.int32),
            pltpu.VMEM((_SG, _CH), jnp.float32),
            pltpu.VMEM((_CH, _D), jnp.float32),
            pltpu.VMEM((_CH, _D), jnp.float32),
            pltpu.VMEM((_CH, _D), jnp.float32),
            pltpu.VMEM_SHARED((_NPAD, _D), jnp.float32),
            pltpu.SemaphoreType.DMA,
            pltpu.SemaphoreType.DMA,
        ],
    )


def _film_body(p0, p1, seqb, wfc, nt, wg, bg, wb, bb, bias, a, out):
    dims = (((1,), (1,)), ((), ()))
    x = p0[...] + p1[...]
    agg = lax.dot_general(x, wfc[...], dims,
                          preferred_element_type=jnp.float32)
    fts = lax.dot_general(seqb[...], wfc[...], dims,
                          preferred_element_type=jnp.float32)
    is0 = nt[...] == 0
    gam = jnp.where(is0, wg[0:1, :], wg[1:2, :]) + bg[...]
    bet = jnp.where(is0, wb[0:1, :], wb[1:2, :]) + bb[...]
    y = gam * agg + bet + bias[...] + fts
    aa = a[0, 0]
    out[...] = jnp.where(y >= 0.0, y, aa * y)


_BM = 1000


def _make_film():
    nb = _N // _BM
    row_spec = pl.BlockSpec((_BM, _D), lambda i: (i, 0))
    full = lambda shape: pl.BlockSpec(shape, lambda i: (0, 0))
    return pl.pallas_call(
        _film_body,
        grid=(nb,),
        in_specs=[
            row_spec,
            pl.BlockSpec((_BM, _D), lambda i: (i + nb, 0)),
            row_spec,
            full((_D, _D)),
            pl.BlockSpec((_BM, 1), lambda i: (i, 0)),
            full((_NC, _D)),
            full((1, _D)),
            full((_NC, _D)),
            full((1, _D)),
            full((1, _D)),
            pl.BlockSpec(memory_space=pltpu.SMEM),
        ],
        out_specs=row_spec,
        out_shape=jax.ShapeDtypeStruct((_N, _D), jnp.float32),
    )


def kernel(seq, edge_index, edge_weight, node_type, W_fc, W_gamma, b_gamma,
           W_beta, b_beta, bias, prelu_a):
    src = edge_index[0].reshape(_NW, _NSG, _SG, _CH)
    dst = edge_index[1].reshape(_NW, _NSG, _SG, _CH)
    w2 = edge_weight.reshape(_NW, _NSG, _SG, _CH)

    partials = _make_spmm()(seq, src, dst, w2)

    return _make_film()(
        partials, partials, seq, W_fc,
        node_type.reshape(_N, 1),
        W_gamma.T, b_gamma.reshape(1, _D),
        W_beta.T, b_beta.reshape(1, _D),
        bias.reshape(1, _D),
        prelu_a.reshape(1, 1),
    )


# speedup vs baseline: 1.7337x; 1.0012x over previous
import jax
import jax.numpy as jnp
from jax import lax
from jax.experimental import pallas as pl
from jax.experimental.pallas import tpu as pltpu
from jax.experimental.pallas import tpu_sc as plsc

_N = 10000
_E = 320000
_D = 128
_NC = 2
_NS = 16
_NW = _NC * _NS
_CH = 80
_EPW = _E // _NW
_NCHUNK = _EPW // _CH
_SG = 25
_NSG = _NCHUNK // _SG
_RPT = 640
_NPAD = _RPT * _NS
_ZR = 80


def _spmm_body(seq_hbm, src_hbm, dst_hbm, w_hbm, out_hbm,
               src_v, dst_v, w_v, rows0, rows1, rows2, acc_sh, gsem, ssem):
    cid = lax.axis_index("c")
    sid = lax.axis_index("s")
    wid = cid * _NS + sid
    bufs = (rows0, rows1, rows2)

    def _zrow(i, carry):
        for k in range(_D // 16):
            rows0[i, pl.ds(k * 16, 16)] = jnp.zeros((16,), jnp.float32)
        return carry
    lax.fori_loop(0, _ZR, _zrow, None)
    r0 = sid * _RPT
    for t in range(_RPT // _ZR):
        pltpu.sync_copy(rows0, acc_sh.at[pl.ds(r0 + t * _ZR, _ZR), :])
    plsc.subcore_barrier()

    def _gather(j, b):
        pltpu.async_copy(seq_hbm.at[src_v.at[j]], bufs[b], gsem)

    def _wait_gather(j, b):
        pltpu.make_async_copy(seq_hbm.at[src_v.at[j]], bufs[b], gsem).wait()

    def _scatter(j, b):
        pltpu.async_copy(bufs[b], acc_sh.at[dst_v.at[j]], ssem, add=True)

    def _wait_scatter(j, b):
        pltpu.make_async_copy(bufs[b], acc_sh.at[dst_v.at[j]], ssem).wait()

    def _scale(j, b):
        rows_b = bufs[b]

        def _g16(i, c2):
            wv = w_v[j, pl.ds(i * 16, 16)]
            for jj in range(16):
                sp = wv.at[jnp.full((16,), jj, jnp.int32)].get(
                    mode="promise_in_bounds")
                e = i * 16 + jj
                for k in range(_D // 16):
                    rows_b[e, pl.ds(k * 16, 16)] = (
                        rows_b[e, pl.ds(k * 16, 16)] * sp)
            return c2
        lax.fori_loop(0, _CH // 16, _g16, None)

    def _group(s, carry):
        pltpu.sync_copy(src_hbm.at[wid, s], src_v)
        pltpu.sync_copy(dst_hbm.at[wid, s], dst_v)
        pltpu.sync_copy(w_hbm.at[wid, s], w_v)
        _gather(0, 0)
        _gather(1, 1)

        def _triple(t, c1):
            for jj in range(3):
                j = 3 * t + jj
                nb = (jj + 2) % 3

                @pl.when(j >= 1)
                def _():
                    _wait_scatter(j - 1, nb)
                _gather(j + 2, nb)
                _wait_gather(j, jj)
                _scale(j, jj)
                _scatter(j, jj)
            return c1
        lax.fori_loop(0, (_SG - 4) // 3, _triple, None)

        for j in range(_SG - 4, _SG):
            b = j % 3
            if j + 2 < _SG:
                _wait_scatter(j - 1, (j + 2) % 3)
                _gather(j + 2, (j + 2) % 3)
            _wait_gather(j, b)
            _scale(j, b)
            _scatter(j, b)
        for j in range(_SG - 3, _SG):
            _wait_scatter(j, j % 3)
        return carry
    lax.fori_loop(0, _NSG, _group, None)

    plsc.subcore_barrier()

    @pl.when(sid < _NS - 1)
    def _dump_full():
        pltpu.sync_copy(acc_sh.at[pl.ds(r0, _RPT), :],
                        out_hbm.at[pl.ds(cid * _N + r0, _RPT), :])

    @pl.when(sid == _NS - 1)
    def _dump_tail():
        rem = _N - (_NS - 1) * _RPT
        pltpu.sync_copy(acc_sh.at[pl.ds(r0, rem), :],
                        out_hbm.at[pl.ds(cid * _N + r0, rem), :])


def _make_spmm():
    mesh = plsc.VectorSubcoreMesh(core_axis_name="c", subcore_axis_name="s")
    return pl.kernel(
        _spmm_body,
        out_type=jax.ShapeDtypeStruct((_NC * _N, _D), jnp.float32),
        mesh=mesh,
        scratch_types=[
            pltpu.VMEM((_SG, _CH), jnp.int32),
            pltpu.VMEM((_SG, _CH), jnp.int32),
            pltpu.VMEM((_SG, _CH), jnp.float32),
            pltpu.VMEM((_CH, _D), jnp.float32),
            pltpu.VMEM((_CH, _D), jnp.float32),
            pltpu.VMEM((_CH, _D), jnp.float32),
            pltpu.VMEM_SHARED((_NPAD, _D), jnp.float32),
            pltpu.SemaphoreType.DMA,
            pltpu.SemaphoreType.DMA,
        ],
    )


def _film_body(p0, p1, seqb, wfc, nt, wg, bg, wb, bb, bias, a, out):
    dims = (((1,), (1,)), ((), ()))
    x = p0[...] + p1[...]
    agg = lax.dot_general(x, wfc[...], dims,
                          preferred_element_type=jnp.float32)
    fts = lax.dot_general(seqb[...], wfc[...], dims,
                          preferred_element_type=jnp.float32)
    is0 = nt[...] == 0
    gam = jnp.where(is0, wg[0:1, :], wg[1:2, :]) + bg[...]
    bet = jnp.where(is0, wb[0:1, :], wb[1:2, :]) + bb[...]
    y = gam * agg + bet + bias[...] + fts
    aa = a[0, 0]
    out[...] = jnp.where(y >= 0.0, y, aa * y)


_BM = 1000


def _make_film():
    nb = _N // _BM
    row_spec = pl.BlockSpec((_BM, _D), lambda i: (i, 0))
    full = lambda shape: pl.BlockSpec(shape, lambda i: (0, 0))
    return pl.pallas_call(
        _film_body,
        grid=(nb,),
        in_specs=[
            row_spec,
            pl.BlockSpec((_BM, _D), lambda i: (i + nb, 0)),
            row_spec,
            full((_D, _D)),
            pl.BlockSpec((_BM, 1), lambda i: (i, 0)),
            full((_NC, _D)),
            full((1, _D)),
            full((_NC, _D)),
            full((1, _D)),
            full((1, _D)),
            pl.BlockSpec(memory_space=pltpu.SMEM),
        ],
        out_specs=row_spec,
        out_shape=jax.ShapeDtypeStruct((_N, _D), jnp.float32),
    )


def kernel(seq, edge_index, edge_weight, node_type, W_fc, W_gamma, b_gamma,
           W_beta, b_beta, bias, prelu_a):
    src = edge_index[0].reshape(_NW, _NSG, _SG, _CH)
    dst = edge_index[1].reshape(_NW, _NSG, _SG, _CH)
    w2 = edge_weight.reshape(_NW, _NSG, _SG, _CH)

    partials = _make_spmm()(seq, src, dst, w2)

    return _make_film()(
        partials, partials, seq, W_fc,
        node_type.reshape(_N, 1),
        W_gamma.T, b_gamma.reshape(1, _D),
        W_beta.T, b_beta.reshape(1, _D),
        bias.reshape(1, _D),
        prelu_a.reshape(1, 1),
    )
